# Initial kernel scaffold; baseline (speedup 1.0000x reference)
import functools

import jax
import jax.numpy as jnp
from jax import lax
from jax.experimental import pallas as pl
from jax.experimental.pallas import tpu as pltpu
from jax.experimental.pallas import tpu_sc as plsc

_BLK = 2000
_K = 1000
_NSUB = 16
_NCORE = 2


def _embed_body(x_ref, w_ref, b_ref, out_ref):
    z = jnp.dot(x_ref[...], w_ref[...], preferred_element_type=jnp.float32)
    z = z + b_ref[...]
    hd = z.shape[1] // 2
    out_ref[0] = z[:, :hd]
    out_ref[1] = z[:, hd:]


def _embed(x, w, b2d):
    n, f = x.shape
    d = w.shape[1]
    grid = n // _BLK
    return pl.pallas_call(
        _embed_body,
        grid=(grid,),
        in_specs=[
            pl.BlockSpec((_BLK, f), lambda i: (i, 0)),
            pl.BlockSpec((f, d), lambda i: (0, 0)),
            pl.BlockSpec((1, d), lambda i: (0, 0)),
        ],
        out_specs=pl.BlockSpec((2, _BLK, d // 2), lambda i: (0, i, 0)),
        out_shape=jax.ShapeDtypeStruct((2, n, d // 2), jnp.float32),
    )(x, w, b2d)


def _sc_segment_sums(zs2, src2, dst, edge_attr, zero16, zero4):
    n2, hd = zs2.shape
    n = n2 // 2
    e = dst.shape[0]
    de = edge_attr.shape[1]
    ea_per_s = e // _NSUB
    eb_per_w = e // (_NSUB * _NCORE)
    nr = n // _NSUB

    mesh = plsc.VectorSubcoreMesh(core_axis_name="c", subcore_axis_name="s")

    @functools.partial(
        pl.kernel,
        out_type=[
            jax.ShapeDtypeStruct((n2, hd), jnp.float32),
            jax.ShapeDtypeStruct((n2, de), jnp.float32),
        ],
        mesh=mesh,
        scratch_types=[
            pltpu.VMEM_SHARED((n, hd), jnp.float32),
            pltpu.VMEM_SHARED((n, de), jnp.float32),
            pltpu.VMEM((_K,), jnp.int32),
            pltpu.VMEM((_K,), jnp.int32),
            pltpu.VMEM((_K, hd), jnp.float32),
            pltpu.VMEM((_K, de), jnp.float32),
            pltpu.SemaphoreType.DMA,
        ],
    )
    def sc_kernel(zs_hbm, src2_hbm, dst_hbm, attr_hbm, z16_hbm, z4_hbm,
                  outa_hbm, outb_hbm,
                  acc_a, acc_b, gidx_v, sidx_v, rows_v, attr_v, sem):
        c = lax.axis_index("c")
        s = lax.axis_index("s")

        pltpu.sync_copy(z16_hbm.at[pl.ds(s * nr, nr)],
                        acc_a.at[pl.ds(s * nr, nr)])
        pltpu.sync_copy(z4_hbm.at[pl.ds(s * nr, nr)],
                        acc_b.at[pl.ds(s * nr, nr)])
        plsc.subcore_barrier()

        base_a = s * ea_per_s

        def a_body(i, carry):
            off = base_a + i * _K
            pltpu.sync_copy(src2_hbm.at[pl.ds(c * e + off, _K)], gidx_v)
            pltpu.sync_copy(dst_hbm.at[pl.ds(off, _K)], sidx_v)
            pltpu.async_copy(zs_hbm.at[gidx_v], rows_v, sem).wait()
            pltpu.sync_copy(rows_v, acc_a.at[sidx_v], add=True)
            return carry

        lax.fori_loop(0, ea_per_s // _K, a_body, 0)

        base_b = c * (e // 2) + s * eb_per_w

        def b_body(i, carry):
            off = base_b + i * _K
            pltpu.sync_copy(attr_hbm.at[pl.ds(off, _K)], attr_v)
            pltpu.sync_copy(dst_hbm.at[pl.ds(off, _K)], sidx_v)
            pltpu.sync_copy(attr_v, acc_b.at[sidx_v], add=True)
            return carry

        lax.fori_loop(0, eb_per_w // _K, b_body, 0)

        plsc.subcore_barrier()

        pltpu.sync_copy(acc_a.at[pl.ds(s * nr, nr)],
                        outa_hbm.at[pl.ds(c * n + s * nr, nr)])
        pltpu.sync_copy(acc_b.at[pl.ds(s * nr, nr)],
                        outb_hbm.at[pl.ds(c * n + s * nr, nr)])

    return sc_kernel(zs2, src2, dst, edge_attr, zero16, zero4)


def _final_body(zs_ref, as_ref, bs_ref,
                wself_ref, wneigh_ref, wedge_ref, bh_ref,
                wselfs_ref, wneighs_ref, wedges_ref, bs2_ref,
                s_out_ref, pool_ref):
    z = jnp.concatenate([zs_ref[0], zs_ref[1]], axis=1)
    a = jnp.concatenate([as_ref[0], as_ref[1]], axis=1)
    b = bs_ref[0] + bs_ref[1]
    h = (jnp.dot(z, wself_ref[...], preferred_element_type=jnp.float32)
         + jnp.dot(a, wneigh_ref[...], preferred_element_type=jnp.float32)
         + jnp.dot(b, wedge_ref[...], preferred_element_type=jnp.float32)
         + bh_ref[...])
    h = jnp.maximum(h, 0.0)
    sl = (jnp.dot(z, wselfs_ref[...], preferred_element_type=jnp.float32)
          + jnp.dot(a, wneighs_ref[...], preferred_element_type=jnp.float32)
          + jnp.dot(b, wedges_ref[...], preferred_element_type=jnp.float32)
          + bs2_ref[...])
    m = jnp.max(sl, axis=1, keepdims=True)
    ex = jnp.exp(sl - m)
    s = ex / jnp.sum(ex, axis=1, keepdims=True)
    s_out_ref[...] = s
    part = lax.dot_general(s, h, (((0,), (0,)), ((), ())),
                           preferred_element_type=jnp.float32)

    @pl.when(pl.program_id(0) == 0)
    def _():
        pool_ref[...] = jnp.zeros_like(pool_ref)

    pool_ref[...] += part


def _final(zs, as_, bs, wself, wneigh, wedge, bh2, wselfs, wneighs, wedges,
           bs2):
    _, n, hd = zs.shape
    de = bs.shape[2]
    d = 2 * hd
    hdim = wself.shape[1]
    c = wselfs.shape[1]
    grid = n // _BLK
    return pl.pallas_call(
        _final_body,
        grid=(grid,),
        in_specs=[
            pl.BlockSpec((2, _BLK, hd), lambda i: (0, i, 0)),
            pl.BlockSpec((2, _BLK, hd), lambda i: (0, i, 0)),
            pl.BlockSpec((2, _BLK, de), lambda i: (0, i, 0)),
            pl.BlockSpec((d, hdim), lambda i: (0, 0)),
            pl.BlockSpec((d, hdim), lambda i: (0, 0)),
            pl.BlockSpec((de, hdim), lambda i: (0, 0)),
            pl.BlockSpec((1, hdim), lambda i: (0, 0)),
            pl.BlockSpec((d, c), lambda i: (0, 0)),
            pl.BlockSpec((d, c), lambda i: (0, 0)),
            pl.BlockSpec((de, c), lambda i: (0, 0)),
            pl.BlockSpec((1, c), lambda i: (0, 0)),
        ],
        out_specs=[
            pl.BlockSpec((_BLK, c), lambda i: (i, 0)),
            pl.BlockSpec((c, hdim), lambda i: (0, 0)),
        ],
        out_shape=[
            jax.ShapeDtypeStruct((n, c), jnp.float32),
            jax.ShapeDtypeStruct((c, hdim), jnp.float32),
        ],
    )(zs, as_, bs, wself, wneigh, wedge, bh2, wselfs, wneighs, wedges, bs2)


def kernel(x_note, edge_index, edge_attr, W_embed, b_embed,
           W_self, W_neigh, W_edge, b_h,
           W_self_s, W_neigh_s, W_edge_s, b_s):
    n = x_note.shape[0]
    ei = edge_index.astype(jnp.int32)
    src = ei[0]
    dst = ei[1]
    src2 = jnp.concatenate([src, src + n])

    zs = _embed(x_note.astype(jnp.float32), W_embed, b_embed.reshape(1, -1))
    zs2 = zs.reshape(2 * n, zs.shape[2])

    zero16 = jnp.zeros((n, zs.shape[2]), jnp.float32)
    zero4 = jnp.zeros((n, edge_attr.shape[1]), jnp.float32)
    out_a, out_b = _sc_segment_sums(zs2, src2, dst, edge_attr, zero16, zero4)

    as_ = out_a.reshape(2, n, zs.shape[2])
    bs = out_b.reshape(2, n, edge_attr.shape[1])

    s_1, x_pool = _final(zs, as_, bs, W_self, W_neigh, W_edge,
                         b_h.reshape(1, -1), W_self_s, W_neigh_s, W_edge_s,
                         b_s.reshape(1, -1))
    return (x_pool, s_1)


# speedup vs baseline: 4.9272x; 4.9272x over previous
import functools

import jax
import jax.numpy as jnp
from jax import lax
from jax.experimental import pallas as pl
from jax.experimental.pallas import tpu as pltpu
from jax.experimental.pallas import tpu_sc as plsc

_BLK = 2000
_K = 1000
_NSUB = 16
_NCORE = 2


def _embed_body(x_ref, w_ref, b_ref, out_ref):
    z = jnp.dot(x_ref[...], w_ref[...], preferred_element_type=jnp.float32)
    z = z + b_ref[...]
    hd = z.shape[1] // 2
    out_ref[0] = z[:, :hd]
    out_ref[1] = z[:, hd:]


def _embed(x, w, b2d):
    n, f = x.shape
    d = w.shape[1]
    grid = n // _BLK
    return pl.pallas_call(
        _embed_body,
        grid=(grid,),
        in_specs=[
            pl.BlockSpec((_BLK, f), lambda i: (i, 0)),
            pl.BlockSpec((f, d), lambda i: (0, 0)),
            pl.BlockSpec((1, d), lambda i: (0, 0)),
        ],
        out_specs=pl.BlockSpec((2, _BLK, d // 2), lambda i: (0, i, 0)),
        out_shape=jax.ShapeDtypeStruct((2, n, d // 2), jnp.float32),
    )(x, w, b2d)


_SC_MESH = dict(core_axis_name="c", subcore_axis_name="s")


def _sc_segment_sum_a(zs2, src2, dst, zero16):
    _, hd = zs2.shape
    npad = zero16.shape[0]
    e = dst.shape[0]
    ea_per_s = e // _NSUB
    nr = npad // _NSUB

    @functools.partial(
        pl.kernel,
        out_type=jax.ShapeDtypeStruct((2 * npad, hd), jnp.float32),
        mesh=plsc.VectorSubcoreMesh(**_SC_MESH),
        compiler_params=pltpu.CompilerParams(use_tc_tiling_on_sc=False),
        scratch_types=[
            pltpu.VMEM_SHARED((npad, hd), jnp.float32),
            pltpu.VMEM((_K,), jnp.int32),
            pltpu.VMEM((_K,), jnp.int32),
            pltpu.VMEM((_K, hd), jnp.float32),
            pltpu.SemaphoreType.DMA,
        ],
    )
    def sc_kernel(zs_hbm, src2_hbm, dst_hbm, z16_hbm, outa_hbm,
                  acc_a, gidx_v, sidx_v, rows_v, sem):
        c = lax.axis_index("c")
        s = lax.axis_index("s")

        pltpu.sync_copy(z16_hbm.at[pl.ds(s * nr, nr)],
                        acc_a.at[pl.ds(s * nr, nr)])
        plsc.subcore_barrier()

        base_a = s * ea_per_s

        def a_body(i, carry):
            off = base_a + i * _K
            pltpu.sync_copy(src2_hbm.at[pl.ds(c * e + off, _K)], gidx_v)
            pltpu.sync_copy(dst_hbm.at[pl.ds(off, _K)], sidx_v)
            pltpu.async_copy(zs_hbm.at[gidx_v], rows_v, sem).wait()
            pltpu.sync_copy(rows_v, acc_a.at[sidx_v], add=True)
            return carry

        lax.fori_loop(0, ea_per_s // _K, a_body, 0)
        plsc.subcore_barrier()
        pltpu.sync_copy(acc_a.at[pl.ds(s * nr, nr)],
                        outa_hbm.at[pl.ds(c * npad + s * nr, nr)])

    return sc_kernel(zs2, src2, dst, zero16)


_KB = 1000


def _sc_segment_sum_b(dst, edge_attr, zero4):
    npad = zero4.shape[0]
    e = dst.shape[0]
    de = edge_attr.shape[1]
    eb_per_w = e // (_NSUB * _NCORE)
    nr = npad // _NSUB

    @functools.partial(
        pl.kernel,
        out_type=jax.ShapeDtypeStruct((2 * npad, de), jnp.float32),
        mesh=plsc.VectorSubcoreMesh(**_SC_MESH),
        compiler_params=pltpu.CompilerParams(use_tc_tiling_on_sc=False),
        scratch_types=[
            pltpu.VMEM_SHARED((npad, de), jnp.float32),
            pltpu.VMEM((_KB,), jnp.int32),
            pltpu.VMEM((_KB, de), jnp.float32),
        ],
    )
    def sc_kernel(dst_hbm, attr_hbm, z4_hbm, outb_hbm,
                  acc_b, sidx_v, attr_v):
        c = lax.axis_index("c")
        s = lax.axis_index("s")

        pltpu.sync_copy(z4_hbm.at[pl.ds(s * nr, nr)],
                        acc_b.at[pl.ds(s * nr, nr)])
        plsc.subcore_barrier()

        base_b = c * (e // 2) + s * eb_per_w

        def b_body(i, carry):
            off = base_b + i * _KB
            pltpu.sync_copy(attr_hbm.at[pl.ds(off, _KB)], attr_v)
            pltpu.sync_copy(dst_hbm.at[pl.ds(off, _KB)], sidx_v)
            pltpu.sync_copy(attr_v, acc_b.at[sidx_v], add=True)
            return carry

        lax.fori_loop(0, eb_per_w // _KB, b_body, 0)
        plsc.subcore_barrier()
        pltpu.sync_copy(acc_b.at[pl.ds(s * nr, nr)],
                        outb_hbm.at[pl.ds(c * npad + s * nr, nr)])

    return sc_kernel(dst, edge_attr, zero4)


def _final_body(zs_ref, as_ref, bs_ref,
                wself_ref, wneigh_ref, wedge_ref, bh_ref,
                wselfs_ref, wneighs_ref, wedges_ref, bs2_ref,
                s_out_ref, pool_ref):
    z = jnp.concatenate([zs_ref[0], zs_ref[1]], axis=1)
    a = jnp.concatenate([as_ref[0], as_ref[1]], axis=1)
    b = bs_ref[0] + bs_ref[1]
    h = (jnp.dot(z, wself_ref[...], preferred_element_type=jnp.float32)
         + jnp.dot(a, wneigh_ref[...], preferred_element_type=jnp.float32)
         + jnp.dot(b, wedge_ref[...], preferred_element_type=jnp.float32)
         + bh_ref[...])
    h = jnp.maximum(h, 0.0)
    sl = (jnp.dot(z, wselfs_ref[...], preferred_element_type=jnp.float32)
          + jnp.dot(a, wneighs_ref[...], preferred_element_type=jnp.float32)
          + jnp.dot(b, wedges_ref[...], preferred_element_type=jnp.float32)
          + bs2_ref[...])
    m = jnp.max(sl, axis=1, keepdims=True)
    ex = jnp.exp(sl - m)
    s = ex / jnp.sum(ex, axis=1, keepdims=True)
    s_out_ref[...] = s
    part = lax.dot_general(s, h, (((0,), (0,)), ((), ())),
                           preferred_element_type=jnp.float32)

    @pl.when(pl.program_id(0) == 0)
    def _():
        pool_ref[...] = jnp.zeros_like(pool_ref)

    pool_ref[...] += part


def _final(zs, as_, bs, wself, wneigh, wedge, bh2, wselfs, wneighs, wedges,
           bs2):
    _, n, hd = zs.shape
    de = bs.shape[2]
    d = 2 * hd
    hdim = wself.shape[1]
    c = wselfs.shape[1]
    grid = n // _BLK
    return pl.pallas_call(
        _final_body,
        grid=(grid,),
        in_specs=[
            pl.BlockSpec((2, _BLK, hd), lambda i: (0, i, 0)),
            pl.BlockSpec((2, _BLK, hd), lambda i: (0, i, 0)),
            pl.BlockSpec((2, _BLK, de), lambda i: (0, i, 0)),
            pl.BlockSpec((d, hdim), lambda i: (0, 0)),
            pl.BlockSpec((d, hdim), lambda i: (0, 0)),
            pl.BlockSpec((de, hdim), lambda i: (0, 0)),
            pl.BlockSpec((1, hdim), lambda i: (0, 0)),
            pl.BlockSpec((d, c), lambda i: (0, 0)),
            pl.BlockSpec((d, c), lambda i: (0, 0)),
            pl.BlockSpec((de, c), lambda i: (0, 0)),
            pl.BlockSpec((1, c), lambda i: (0, 0)),
        ],
        out_specs=[
            pl.BlockSpec((_BLK, c), lambda i: (i, 0)),
            pl.BlockSpec((c, hdim), lambda i: (0, 0)),
        ],
        out_shape=[
            jax.ShapeDtypeStruct((n, c), jnp.float32),
            jax.ShapeDtypeStruct((c, hdim), jnp.float32),
        ],
    )(zs, as_, bs, wself, wneigh, wedge, bh2, wselfs, wneighs, wedges, bs2)


def kernel(x_note, edge_index, edge_attr, W_embed, b_embed,
           W_self, W_neigh, W_edge, b_h,
           W_self_s, W_neigh_s, W_edge_s, b_s):
    n = x_note.shape[0]
    ei = edge_index.astype(jnp.int32)
    src = ei[0]
    dst = ei[1]
    src2 = jnp.concatenate([src, src + n])

    zs = _embed(x_note.astype(jnp.float32), W_embed, b_embed.reshape(1, -1))
    zs2 = zs.reshape(2 * n, zs.shape[2])

    npad = ((n + 127) // 128) * 128
    attr8 = jnp.pad(edge_attr.astype(jnp.float32), ((0, 0), (0, 4)))
    w_edge8 = jnp.pad(W_edge, ((0, 4), (0, 0)))
    w_edge_s8 = jnp.pad(W_edge_s, ((0, 4), (0, 0)))
    zero16 = jnp.zeros((npad, zs.shape[2]), jnp.float32)
    zero8 = jnp.zeros((npad, attr8.shape[1]), jnp.float32)
    out_b = _sc_segment_sum_b(dst, attr8, zero8)
    out_a = _sc_segment_sum_a(zs2, src2, dst, zero16)

    as_ = out_a.reshape(2, npad, zs.shape[2])
    bs = out_b.reshape(2, npad, attr8.shape[1])

    s_1, x_pool = _final(zs, as_, bs, W_self, W_neigh, w_edge8,
                         b_h.reshape(1, -1), W_self_s, W_neigh_s, w_edge_s8,
                         b_s.reshape(1, -1))
    return (x_pool, s_1)
